# baseline (speedup 1.0000x reference)
import functools

import jax
import jax.numpy as jnp
from jax import lax
from jax.experimental import pallas as pl
from jax.experimental.pallas import tpu as pltpu
from jax.experimental.pallas import tpu_sc as plsc

_NEI = 20
_POOL_NEI = 4


def _norm_cols(d):
    n = jnp.sqrt(jnp.sum(d * d, axis=0, keepdims=True))
    return d / jnp.maximum(n, 1e-12)


def _neg_dist_t(cands, rows_t):
    cx, cy, cz = cands[:, 0:1], cands[:, 1:2], cands[:, 2:3]
    rx, ry, rz = rows_t[0:1, :], rows_t[1:2, :], rows_t[2:3, :]
    inner = cx * rx + cy * ry + cz * rz
    qc = cx * cx + cy * cy + cz * cz
    qr = rx * rx + ry * ry + rz * rz
    return -((-2.0 * inner + qc) + qr)


def _select(neg, col, big):
    m = jnp.max(neg, axis=0, keepdims=True)
    ismax = neg == m
    idx = jnp.min(jnp.where(ismax, col, big), axis=0, keepdims=True)
    oh = col == idx
    return idx, oh, jnp.where(oh, -jnp.inf, neg)


def _dot0(a, b):
    return jax.lax.dot_general(
        a, b, (((0,), (0,)), ((), ())), preferred_element_type=jnp.float32
    )


def _dot(a, b):
    return jax.lax.dot_general(
        a, b, (((1,), (0,)), ((), ())), preferred_element_type=jnp.float32
    )


def _theta(nd3, sd):
    return jnp.maximum(_dot(nd3, sd), 0.0)


def _knn_body(with_surf, flat_idx, *refs):
    if with_surf:
        v_ref, vf_ref, vt_ref, dir0_ref, nidx_ref, nd_ref, fm0_ref = refs
    else:
        v_ref, vf_ref, vt_ref, nidx_ref, nd_ref = refs
    rows, vfull, rows_t = v_ref[0], vf_ref[0], vt_ref[0]
    r, m = rows.shape[0], vfull.shape[0]
    col = jax.lax.broadcasted_iota(jnp.int32, (m, r), 0)
    neg = _neg_dist_t(vfull, rows_t)
    if with_surf:
        sd0 = _norm_cols(dir0_ref[...])
        acc = None
    idxs, nds = [], []
    for t in range(_NEI + 1):
        idx, oh, neg = _select(neg, col, m)
        if t == 0:
            continue
        d3 = _dot0(oh.astype(jnp.float32), vfull) - rows
        den = jnp.maximum(
            jnp.sqrt(jnp.sum(d3 * d3, axis=1, keepdims=True)), 1e-12)
        nd3 = d3 / den
        idxs.append(idx)
        nds.append(nd3)
        if with_surf:
            th = _theta(nd3, sd0)
            acc = th if acc is None else jnp.maximum(acc, th)
    nidx = jnp.concatenate(idxs, axis=0)
    if flat_idx:
        nidx = nidx + pl.program_id(0) * m
    nidx_ref[0] = nidx
    nd_ref[0] = jnp.concatenate(nds, axis=1)
    if with_surf:
        fm0_ref[0] = jnp.maximum(acc, 0.0)


def _conv_body(cout, relu_out, reduce_out, fm_full_ref, fm_chunk_ref, w_ref,
               b_ref, d_ref, nidx_ref, nd_ref, out_ref):
    fm_full, fm_chunk = fm_full_ref[0], fm_chunk_ref[0]
    w, b = w_ref[...], b_ref[...]
    sup = _dot(fm_full, w[:, cout:]) + b[:, cout:]
    center = _dot(fm_chunk, w[:, :cout]) + b[:, :cout]
    sd = _norm_cols(d_ref[...])
    nidx = nidx_ref[0]
    nd = nd_ref[0]
    r, m = fm_chunk.shape[0], fm_full.shape[0]
    col = jax.lax.broadcasted_iota(jnp.int32, (m, r), 0)
    acc = None
    for t in range(_NEI):
        oh = (col == nidx[t:t + 1, :]).astype(jnp.float32)
        g = _dot0(oh, sup)
        v = _theta(nd[:, 3 * t:3 * t + 3], sd) * g
        acc = v if acc is None else jnp.maximum(acc, v)
    res = center + acc
    if relu_out:
        res = jnp.maximum(res, 0.0)
    if reduce_out:
        res = jnp.max(res, axis=0, keepdims=True)
    out_ref[0] = res


_SC_CORES = 2
_SC_SUBCORES = 16


def _sc_gather(table, idx, chunk=256):
    n, c = idx.shape[0], table.shape[1]
    nw = _SC_CORES * _SC_SUBCORES
    b_per_w = n // nw
    nch = b_per_w // chunk
    mesh = plsc.VectorSubcoreMesh(
        core_axis_name="c", subcore_axis_name="s",
        num_cores=_SC_CORES, num_subcores=_SC_SUBCORES)

    @functools.partial(
        pl.kernel, mesh=mesh,
        out_type=jax.ShapeDtypeStruct((n, c), jnp.float32),
        scratch_types=[pltpu.VMEM((b_per_w,), jnp.int32),
                       pltpu.VMEM((2, chunk, c), jnp.float32),
                       pltpu.SemaphoreType.DMA,
                       pltpu.SemaphoreType.DMA],
    )
    def k(table_hbm, idx_hbm, out_hbm, idx_v, rows_v, sem0, sem1):
        wid = lax.axis_index("s") * _SC_CORES + lax.axis_index("c")
        base = wid * b_per_w
        pltpu.sync_copy(idx_hbm.at[pl.ds(base, b_per_w)], idx_v)
        sems = (sem0, sem1)
        handles = [None, None]
        for ci in range(nch):
            s = ci % 2
            handles[s] = pltpu.async_copy(
                table_hbm.at[idx_v.at[pl.ds(ci * chunk, chunk)]],
                rows_v.at[s], sems[s])
            if ci > 0:
                handles[1 - s].wait()
                pltpu.sync_copy(
                    rows_v.at[1 - s],
                    out_hbm.at[pl.ds(base + (ci - 1) * chunk, chunk)])
        last = (nch - 1) % 2
        handles[last].wait()
        pltpu.sync_copy(
            rows_v.at[last],
            out_hbm.at[pl.ds(base + (nch - 1) * chunk, chunk)])

    return k(table, idx)


def _feat_body(fm_ref, w_ref, b_ref, out_ref):
    out_ref[0] = _dot(fm_ref[0], w_ref[...]) + b_ref[...]


def _convg_body(cout, relu_out, feat_chunk_ref, d_ref, g_ref, nd_ref,
                out_ref):
    feat = feat_chunk_ref[0]
    sd = _norm_cols(d_ref[...])
    g = g_ref[0]
    nd = nd_ref[0]
    acc = None
    for t in range(_NEI):
        v = _theta(nd[:, 3 * t:3 * t + 3], sd) * g[t][:, cout:]
        acc = v if acc is None else jnp.maximum(acc, v)
    res = feat[:, :cout] + acc
    if relu_out:
        res = jnp.maximum(res, 0.0)
    out_ref[0] = res


def _pool_body(svt_ref, vf_ref, fm_ref, out_ref):
    svt, vfull, fm = svt_ref[0], vf_ref[0], fm_ref[0]
    m, r = vfull.shape[0], svt.shape[1]
    col = jax.lax.broadcasted_iota(jnp.int32, (m, r), 0)
    neg = _neg_dist_t(vfull, svt)
    acc = None
    for t in range(_POOL_NEI + 1):
        _, oh, neg = _select(neg, col, m)
        if t == 0:
            continue
        g = _dot0(oh.astype(jnp.float32), fm)
        acc = g if acc is None else jnp.maximum(acc, g)
    out_ref[0] = acc


def _head_body(fg_ref, w1_ref, b1_ref, g_ref, be_ref, m_ref, var_ref,
               w2_ref, b2_ref, out_ref):
    h = _dot(fg_ref[...], w1_ref[...]) + b1_ref[...]
    h = (h - m_ref[...]) / jnp.sqrt(var_ref[...] + 1e-5) * g_ref[...] + be_ref[...]
    h = jnp.maximum(h, 0.0)
    out_ref[...] = _dot(h, w2_ref[...]) + b2_ref[...]


def _chunk(shape):
    return pl.BlockSpec((1,) + shape, lambda b, c: (b, c) + (0,) * (len(shape) - 1))


def _bfull(shape):
    return pl.BlockSpec((1,) + shape, lambda b, c: (b,) + (0,) * len(shape))


def _wfull(shape):
    return pl.BlockSpec(shape, lambda b, c: (0,) * len(shape))


def _chunk_t(shape):
    return pl.BlockSpec((1,) + shape, lambda b, c: (b, 0, c))


def _run_knn(v, vt, nchunks, dir0=None, flat_idx=False):
    bsz, nverts, _ = v.shape
    rc = nverts // nchunks
    f32, i32 = jnp.float32, jnp.int32
    out_shape = [jax.ShapeDtypeStruct((bsz, _NEI, nverts), i32),
                 jax.ShapeDtypeStruct((bsz, nverts, 3 * _NEI), f32)]
    out_specs = [_chunk_t((_NEI, rc)), _chunk((rc, 3 * _NEI))]
    in_specs = [_chunk((rc, 3)), _bfull((nverts, 3)), _chunk_t((3, rc))]
    args = [v, v, vt]
    if dir0 is not None:
        in_specs.append(_wfull((3, 32)))
        args.append(dir0)
        out_shape.append(jax.ShapeDtypeStruct((bsz, nverts, 32), f32))
        out_specs.append(_chunk((rc, 32)))
    return pl.pallas_call(
        functools.partial(_knn_body, dir0 is not None, flat_idx),
        grid=(bsz, nchunks),
        in_specs=in_specs,
        out_specs=out_specs,
        out_shape=out_shape,
    )(*args)


def _run_conv(fm, w, b, d, nidx, nd, cout, nchunks,
              relu_out=True, reduce_out=False):
    bsz, nverts, cin = fm.shape
    rc = nverts // nchunks
    out_r = 1 if reduce_out else rc
    out_n = 1 if reduce_out else nverts
    return pl.pallas_call(
        functools.partial(_conv_body, cout, relu_out, reduce_out),
        grid=(bsz, nchunks),
        in_specs=[_bfull((nverts, cin)), _chunk((rc, cin)),
                  _wfull((cin, 2 * cout)), _wfull((1, 2 * cout)),
                  _wfull((3, cout)), _chunk_t((_NEI, rc)),
                  _chunk((rc, 3 * _NEI))],
        out_specs=_chunk((out_r, cout)),
        out_shape=jax.ShapeDtypeStruct((bsz, out_n, cout), jnp.float32),
    )(fm, fm, w, b.reshape(1, -1), d, nidx, nd)


def _run_pool(svt, vfull, fm):
    bsz, r = svt.shape[0], svt.shape[2]
    nverts, c = fm.shape[1], fm.shape[2]
    return pl.pallas_call(
        _pool_body,
        grid=(bsz, 1),
        in_specs=[_bfull((3, r)), _bfull((nverts, 3)), _bfull((nverts, c))],
        out_specs=_bfull((r, c)),
        out_shape=jax.ShapeDtypeStruct((bsz, r, c), jnp.float32),
    )(svt, vfull, fm)


def kernel(vertices, dir0, w1, b1, d1, w2, b2, d2, w3, b3, d3, w4, b4, d4,
           fc1_w, fc1_b, bn_g, bn_b, bn_m, bn_v, fc2_w, fc2_b):
    bsz = vertices.shape[0]
    v = jnp.transpose(vertices, (0, 2, 1))
    vt = vertices

    samp1 = jax.random.permutation(jax.random.key(100), 1024)[:256]
    samp2 = jax.random.permutation(jax.random.key(101), 256)[:64]
    sv1 = v[:, samp1, :]
    sv1t = jnp.transpose(sv1, (0, 2, 1))
    sv2 = sv1[:, samp2, :]
    sv2t = jnp.transpose(sv2, (0, 2, 1))

    nidxf, nd, fm0 = _run_knn(v, vt, 2, dir0=dir0, flat_idx=True)
    feat1 = pl.pallas_call(
        _feat_body,
        grid=(bsz, 1),
        in_specs=[_bfull((1024, 32)), _wfull((32, 128)), _wfull((1, 128))],
        out_specs=_bfull((1024, 128)),
        out_shape=jax.ShapeDtypeStruct((bsz, 1024, 128), jnp.float32),
    )(fm0, w1, b1.reshape(1, -1))
    idx_flat = nidxf.reshape(-1)
    g1 = _sc_gather(feat1.reshape(bsz * 1024, 128), idx_flat)
    g1 = g1.reshape(bsz, _NEI, 1024, 128)
    rc1 = 512
    fm1 = pl.pallas_call(
        functools.partial(_convg_body, 64, True),
        grid=(bsz, 1024 // rc1),
        in_specs=[_chunk((rc1, 128)), _wfull((3, 64)),
                  pl.BlockSpec((1, _NEI, rc1, 128),
                               lambda b, c: (b, 0, c, 0)),
                  _chunk((rc1, 3 * _NEI))],
        out_specs=_chunk((rc1, 64)),
        out_shape=jax.ShapeDtypeStruct((bsz, 1024, 64), jnp.float32),
    )(feat1, d1, g1, nd)
    fm1p = _run_pool(sv1t, v, fm1)

    nidx, nd = _run_knn(sv1, sv1t, 1)
    fm2 = _run_conv(fm1p, w2, b2, d2, nidx, nd, 128, 1)
    fm3 = _run_conv(fm2, w3, b3, d3, nidx, nd, 256, 1)
    fm3p = _run_pool(sv2t, sv1, fm3)

    nidx, nd = _run_knn(sv2, sv2t, 1)
    fg = _run_conv(fm3p, w4, b4, d4, nidx, nd, 1024, 1,
                   relu_out=False, reduce_out=True)

    row = lambda x: x.reshape(1, -1)
    out = pl.pallas_call(
        _head_body,
        in_specs=[pl.BlockSpec((bsz, 1024), lambda: (0, 0)),
                  pl.BlockSpec((1024, 256), lambda: (0, 0))]
        + [pl.BlockSpec((1, 256), lambda: (0, 0))] * 5
        + [pl.BlockSpec((256, 40), lambda: (0, 0)),
           pl.BlockSpec((1, 40), lambda: (0, 0))],
        out_specs=pl.BlockSpec((bsz, 40), lambda: (0, 0)),
        out_shape=jax.ShapeDtypeStruct((bsz, 40), jnp.float32),
    )(fg.reshape(bsz, 1024), fc1_w, row(fc1_b), row(bn_g), row(bn_b),
      row(bn_m), row(bn_v), fc2_w, row(fc2_b))
    return out


# speedup vs baseline: 1.0101x; 1.0101x over previous
import functools

import jax
import jax.numpy as jnp
from jax import lax
from jax.experimental import pallas as pl
from jax.experimental.pallas import tpu as pltpu
from jax.experimental.pallas import tpu_sc as plsc

_NEI = 20
_POOL_NEI = 4


def _norm_cols(d):
    n = jnp.sqrt(jnp.sum(d * d, axis=0, keepdims=True))
    return d / jnp.maximum(n, 1e-12)


def _neg_dist_t(cands, rows_t):
    cx, cy, cz = cands[:, 0:1], cands[:, 1:2], cands[:, 2:3]
    rx, ry, rz = rows_t[0:1, :], rows_t[1:2, :], rows_t[2:3, :]
    inner = cx * rx + cy * ry + cz * rz
    qc = cx * cx + cy * cy + cz * cz
    qr = rx * rx + ry * ry + rz * rz
    return -((-2.0 * inner + qc) + qr)


def _select(neg, col, big):
    m = jnp.max(neg, axis=0, keepdims=True)
    ismax = neg == m
    idx = jnp.min(jnp.where(ismax, col, big), axis=0, keepdims=True)
    oh = col == idx
    return idx, oh, jnp.where(oh, -jnp.inf, neg)


def _dot0(a, b):
    return jax.lax.dot_general(
        a, b, (((0,), (0,)), ((), ())), preferred_element_type=jnp.float32
    )


def _dot(a, b):
    return jax.lax.dot_general(
        a, b, (((1,), (0,)), ((), ())), preferred_element_type=jnp.float32
    )


def _theta(nd3, sd):
    return jnp.maximum(_dot(nd3, sd), 0.0)


def _knn_body(with_surf, flat_idx, *refs):
    if with_surf:
        v_ref, vf_ref, vt_ref, dir0_ref, nidx_ref, nd_ref, fm0_ref = refs
    else:
        v_ref, vf_ref, vt_ref, nidx_ref, nd_ref = refs
    rows, vfull, rows_t = v_ref[0], vf_ref[0], vt_ref[0]
    r, m = rows.shape[0], vfull.shape[0]
    col = jax.lax.broadcasted_iota(jnp.int32, (m, r), 0)
    neg = _neg_dist_t(vfull, rows_t)
    if with_surf:
        sd0 = _norm_cols(dir0_ref[...])
        acc = None
    idxs, nds = [], []
    for t in range(_NEI + 1):
        idx, oh, neg = _select(neg, col, m)
        if t == 0:
            continue
        d3 = _dot0(oh.astype(jnp.float32), vfull) - rows
        den = jnp.maximum(
            jnp.sqrt(jnp.sum(d3 * d3, axis=1, keepdims=True)), 1e-12)
        nd3 = d3 / den
        idxs.append(idx)
        nds.append(nd3)
        if with_surf:
            th = _theta(nd3, sd0)
            acc = th if acc is None else jnp.maximum(acc, th)
    nidx = jnp.concatenate(idxs, axis=0)
    if flat_idx:
        nidx = nidx + pl.program_id(0) * m
    nidx_ref[0] = nidx
    nd_ref[0] = jnp.concatenate(nds, axis=1)
    if with_surf:
        fm0_ref[0] = jnp.maximum(acc, 0.0)


def _conv_body(cout, relu_out, reduce_out, fm_full_ref, fm_chunk_ref, w_ref,
               b_ref, d_ref, nidx_ref, nd_ref, out_ref):
    fm_full, fm_chunk = fm_full_ref[0], fm_chunk_ref[0]
    w, b = w_ref[...], b_ref[...]
    sup = _dot(fm_full, w[:, cout:]) + b[:, cout:]
    center = _dot(fm_chunk, w[:, :cout]) + b[:, :cout]
    sd = _norm_cols(d_ref[...])
    nidx = nidx_ref[0]
    nd = nd_ref[0]
    r, m = fm_chunk.shape[0], fm_full.shape[0]
    col = jax.lax.broadcasted_iota(jnp.int32, (m, r), 0)
    acc = None
    for t in range(_NEI):
        oh = (col == nidx[t:t + 1, :]).astype(jnp.float32)
        g = _dot0(oh, sup)
        v = _theta(nd[:, 3 * t:3 * t + 3], sd) * g
        acc = v if acc is None else jnp.maximum(acc, v)
    res = center + acc
    if relu_out:
        res = jnp.maximum(res, 0.0)
    if reduce_out:
        res = jnp.max(res, axis=0, keepdims=True)
    out_ref[0] = res


_SC_CORES = 2
_SC_SUBCORES = 16


def _sc_gather(table, idx, chunk=512):
    n, c = idx.shape[0], table.shape[1]
    nw = _SC_CORES * _SC_SUBCORES
    b_per_w = n // nw
    nch = b_per_w // chunk
    mesh = plsc.VectorSubcoreMesh(
        core_axis_name="c", subcore_axis_name="s",
        num_cores=_SC_CORES, num_subcores=_SC_SUBCORES)

    @functools.partial(
        pl.kernel, mesh=mesh,
        out_type=jax.ShapeDtypeStruct((n, c), jnp.float32),
        scratch_types=[pltpu.VMEM((chunk,), jnp.int32),
                       pltpu.VMEM((chunk, c), jnp.float32),
                       pltpu.SemaphoreType.DMA],
    )
    def k(table_hbm, idx_hbm, out_hbm, idx_v, rows_v, sem):
        wid = lax.axis_index("s") * _SC_CORES + lax.axis_index("c")
        base = wid * b_per_w
        for ci in range(nch):
            o = base + ci * chunk
            pltpu.sync_copy(idx_hbm.at[pl.ds(o, chunk)], idx_v)
            pltpu.async_copy(table_hbm.at[idx_v], rows_v, sem).wait()
            pltpu.sync_copy(rows_v, out_hbm.at[pl.ds(o, chunk)])

    return k(table, idx)


def _feat_body(fm_ref, w_ref, b_ref, out_ref):
    out_ref[0] = _dot(fm_ref[0], w_ref[...]) + b_ref[...]


def _convg_body(cout, relu_out, feat_chunk_ref, d_ref, g_ref, nd_ref,
                out_ref):
    feat = feat_chunk_ref[0]
    sd = _norm_cols(d_ref[...])
    g = g_ref[0]
    nd = nd_ref[0]
    acc = None
    for t in range(_NEI):
        v = _theta(nd[:, 3 * t:3 * t + 3], sd) * g[t][:, cout:]
        acc = v if acc is None else jnp.maximum(acc, v)
    res = feat[:, :cout] + acc
    if relu_out:
        res = jnp.maximum(res, 0.0)
    out_ref[0] = res


def _pool_body(svt_ref, vf_ref, fm_ref, out_ref):
    svt, vfull, fm = svt_ref[0], vf_ref[0], fm_ref[0]
    m, r = vfull.shape[0], svt.shape[1]
    col = jax.lax.broadcasted_iota(jnp.int32, (m, r), 0)
    neg = _neg_dist_t(vfull, svt)
    acc = None
    for t in range(_POOL_NEI + 1):
        _, oh, neg = _select(neg, col, m)
        if t == 0:
            continue
        g = _dot0(oh.astype(jnp.float32), fm)
        acc = g if acc is None else jnp.maximum(acc, g)
    out_ref[0] = acc


def _head_body(fg_ref, w1_ref, b1_ref, g_ref, be_ref, m_ref, var_ref,
               w2_ref, b2_ref, out_ref):
    h = _dot(fg_ref[...], w1_ref[...]) + b1_ref[...]
    h = (h - m_ref[...]) / jnp.sqrt(var_ref[...] + 1e-5) * g_ref[...] + be_ref[...]
    h = jnp.maximum(h, 0.0)
    out_ref[...] = _dot(h, w2_ref[...]) + b2_ref[...]


def _chunk(shape):
    return pl.BlockSpec((1,) + shape, lambda b, c: (b, c) + (0,) * (len(shape) - 1))


def _bfull(shape):
    return pl.BlockSpec((1,) + shape, lambda b, c: (b,) + (0,) * len(shape))


def _wfull(shape):
    return pl.BlockSpec(shape, lambda b, c: (0,) * len(shape))


def _chunk_t(shape):
    return pl.BlockSpec((1,) + shape, lambda b, c: (b, 0, c))


def _run_knn(v, vt, nchunks, dir0=None, flat_idx=False):
    bsz, nverts, _ = v.shape
    rc = nverts // nchunks
    f32, i32 = jnp.float32, jnp.int32
    out_shape = [jax.ShapeDtypeStruct((bsz, _NEI, nverts), i32),
                 jax.ShapeDtypeStruct((bsz, nverts, 3 * _NEI), f32)]
    out_specs = [_chunk_t((_NEI, rc)), _chunk((rc, 3 * _NEI))]
    in_specs = [_chunk((rc, 3)), _bfull((nverts, 3)), _chunk_t((3, rc))]
    args = [v, v, vt]
    if dir0 is not None:
        in_specs.append(_wfull((3, 32)))
        args.append(dir0)
        out_shape.append(jax.ShapeDtypeStruct((bsz, nverts, 32), f32))
        out_specs.append(_chunk((rc, 32)))
    return pl.pallas_call(
        functools.partial(_knn_body, dir0 is not None, flat_idx),
        grid=(bsz, nchunks),
        in_specs=in_specs,
        out_specs=out_specs,
        out_shape=out_shape,
    )(*args)


def _run_conv(fm, w, b, d, nidx, nd, cout, nchunks,
              relu_out=True, reduce_out=False):
    bsz, nverts, cin = fm.shape
    rc = nverts // nchunks
    out_r = 1 if reduce_out else rc
    out_n = 1 if reduce_out else nverts
    return pl.pallas_call(
        functools.partial(_conv_body, cout, relu_out, reduce_out),
        grid=(bsz, nchunks),
        in_specs=[_bfull((nverts, cin)), _chunk((rc, cin)),
                  _wfull((cin, 2 * cout)), _wfull((1, 2 * cout)),
                  _wfull((3, cout)), _chunk_t((_NEI, rc)),
                  _chunk((rc, 3 * _NEI))],
        out_specs=_chunk((out_r, cout)),
        out_shape=jax.ShapeDtypeStruct((bsz, out_n, cout), jnp.float32),
    )(fm, fm, w, b.reshape(1, -1), d, nidx, nd)


def _run_pool(svt, vfull, fm):
    bsz, r = svt.shape[0], svt.shape[2]
    nverts, c = fm.shape[1], fm.shape[2]
    return pl.pallas_call(
        _pool_body,
        grid=(bsz, 1),
        in_specs=[_bfull((3, r)), _bfull((nverts, 3)), _bfull((nverts, c))],
        out_specs=_bfull((r, c)),
        out_shape=jax.ShapeDtypeStruct((bsz, r, c), jnp.float32),
    )(svt, vfull, fm)


def kernel(vertices, dir0, w1, b1, d1, w2, b2, d2, w3, b3, d3, w4, b4, d4,
           fc1_w, fc1_b, bn_g, bn_b, bn_m, bn_v, fc2_w, fc2_b):
    bsz = vertices.shape[0]
    v = jnp.transpose(vertices, (0, 2, 1))
    vt = vertices

    samp1 = jax.random.permutation(jax.random.key(100), 1024)[:256]
    samp2 = jax.random.permutation(jax.random.key(101), 256)[:64]
    sv1 = v[:, samp1, :]
    sv1t = jnp.transpose(sv1, (0, 2, 1))
    sv2 = sv1[:, samp2, :]
    sv2t = jnp.transpose(sv2, (0, 2, 1))

    nidxf, nd, fm0 = _run_knn(v, vt, 2, dir0=dir0, flat_idx=True)
    feat1 = pl.pallas_call(
        _feat_body,
        grid=(bsz, 1),
        in_specs=[_bfull((1024, 32)), _wfull((32, 128)), _wfull((1, 128))],
        out_specs=_bfull((1024, 128)),
        out_shape=jax.ShapeDtypeStruct((bsz, 1024, 128), jnp.float32),
    )(fm0, w1, b1.reshape(1, -1))
    idx_flat = nidxf.reshape(-1)
    g1 = _sc_gather(feat1.reshape(bsz * 1024, 128), idx_flat)
    g1 = g1.reshape(bsz, _NEI, 1024, 128)
    rc1 = 512
    fm1 = pl.pallas_call(
        functools.partial(_convg_body, 64, True),
        grid=(bsz, 1024 // rc1),
        in_specs=[_chunk((rc1, 128)), _wfull((3, 64)),
                  pl.BlockSpec((1, _NEI, rc1, 128),
                               lambda b, c: (b, 0, c, 0)),
                  _chunk((rc1, 3 * _NEI))],
        out_specs=_chunk((rc1, 64)),
        out_shape=jax.ShapeDtypeStruct((bsz, 1024, 64), jnp.float32),
    )(feat1, d1, g1, nd)
    fm1p = _run_pool(sv1t, v, fm1)

    nidx, nd = _run_knn(sv1, sv1t, 1)
    fm2 = _run_conv(fm1p, w2, b2, d2, nidx, nd, 128, 1)
    fm3 = _run_conv(fm2, w3, b3, d3, nidx, nd, 256, 1)
    fm3p = _run_pool(sv2t, sv1, fm3)

    nidx, nd = _run_knn(sv2, sv2t, 1)
    fg = _run_conv(fm3p, w4, b4, d4, nidx, nd, 1024, 1,
                   relu_out=False, reduce_out=True)

    row = lambda x: x.reshape(1, -1)
    out = pl.pallas_call(
        _head_body,
        in_specs=[pl.BlockSpec((bsz, 1024), lambda: (0, 0)),
                  pl.BlockSpec((1024, 256), lambda: (0, 0))]
        + [pl.BlockSpec((1, 256), lambda: (0, 0))] * 5
        + [pl.BlockSpec((256, 40), lambda: (0, 0)),
           pl.BlockSpec((1, 40), lambda: (0, 0))],
        out_specs=pl.BlockSpec((bsz, 40), lambda: (0, 0)),
        out_shape=jax.ShapeDtypeStruct((bsz, 40), jnp.float32),
    )(fg.reshape(bsz, 1024), fc1_w, row(fc1_b), row(bn_g), row(bn_b),
      row(bn_m), row(bn_v), fc2_w, row(fc2_b))
    return out
